# Initial kernel scaffold; baseline (speedup 1.0000x reference)
import jax
import jax.numpy as jnp
from jax.experimental import pallas as pl


def kernel(output, target):
    raise NotImplementedError("write your pallas kernel here")


# speedup vs baseline: 25.5534x; 25.5534x over previous
import jax
import jax.numpy as jnp
from jax import lax
from jax.experimental import pallas as pl
from jax.experimental.pallas import tpu as pltpu

EPS = 1e-06
M_TOTAL = 8 * 512 * 512
K_KEEP = int(M_TOTAL * 0.75)
ROWS = 2048
COLS = 1024
GRID = 8
BLK = ROWS // GRID
MAX_FINITE_BITS = 0x7F800000


def _focal_body(x_ref, t_ref, out_ref, bits_ref, acc_ref):
    i = pl.program_id(0)
    x = x_ref[...]
    t = t_ref[...]
    posf = (t == 1).astype(jnp.float32)
    negf = 1.0 - posf

    e = jnp.exp(-jnp.abs(x))
    log1pe = jnp.log1p(e)
    ls_pos = jnp.minimum(x, 0.0) - log1pe
    ls_neg = jnp.minimum(-x, 0.0) - log1pe
    sig = jnp.where(x >= 0.0, 1.0 / (1.0 + e), e / (1.0 + e))
    pt = jnp.clip(sig, EPS, 1.0 - EPS)

    pos_loss = -jnp.sqrt(jnp.sqrt(1.0 - pt)) * ls_pos * posf
    neg_loss = jnp.abs(-(pt * pt) * ls_neg * negf)
    bits_ref[pl.ds(i * BLK, BLK), :] = lax.bitcast_convert_type(
        neg_loss, jnp.int32
    )

    p_sum = jnp.sum(pos_loss)
    p_cnt = jnp.sum(posf)
    n_cnt = jnp.sum(negf)

    @pl.when(i == 0)
    def _():
        acc_ref[0] = p_sum
        acc_ref[1] = p_cnt
        acc_ref[2] = n_cnt

    @pl.when(i > 0)
    def _():
        acc_ref[0] += p_sum
        acc_ref[1] += p_cnt
        acc_ref[2] += n_cnt

    @pl.when(i == GRID - 1)
    def _():
        def step(_, carry):
            lo, hi = carry
            mid = lo + (hi - lo) // 2
            cnt = jnp.sum((bits_ref[...] <= mid).astype(jnp.int32))
            ge = cnt >= K_KEEP
            return jnp.where(ge, lo, mid + 1), jnp.where(ge, mid, hi)

        lo, _ = lax.fori_loop(
            0, 31, step, (jnp.int32(0), jnp.int32(MAX_FINITE_BITS))
        )

        b = bits_ref[...]
        vals = lax.bitcast_convert_type(b, jnp.float32)
        mask_lt = b < lo
        cnt_lt = jnp.sum(mask_lt.astype(jnp.int32))
        sum_lt = jnp.sum(jnp.where(mask_lt, vals, 0.0))
        tval = lax.bitcast_convert_type(lo, jnp.float32)
        bottom = sum_lt + (K_KEEP - cnt_lt).astype(jnp.float32) * tval

        loss = acc_ref[0] / (acc_ref[1] + EPS) + bottom / (acc_ref[2] + EPS)
        out_ref[0, 0] = loss


def kernel(output, target):
    x = output.astype(jnp.float32).reshape(ROWS, COLS)
    t = target.reshape(ROWS, COLS)
    res = pl.pallas_call(
        _focal_body,
        grid=(GRID,),
        in_specs=[
            pl.BlockSpec((BLK, COLS), lambda i: (i, 0)),
            pl.BlockSpec((BLK, COLS), lambda i: (i, 0)),
        ],
        out_shape=jax.ShapeDtypeStruct((1, 1), jnp.float32),
        out_specs=pl.BlockSpec(memory_space=pltpu.SMEM),
        scratch_shapes=[
            pltpu.VMEM((ROWS, COLS), jnp.int32),
            pltpu.SMEM((3,), jnp.float32),
        ],
    )(x, t)
    return res[0, 0]
